# baseline (speedup 1.0000x reference)
import functools

import jax
import jax.numpy as jnp
from jax import lax
from jax.experimental import pallas as pl
from jax.experimental.pallas import tpu as pltpu
from jax.experimental.pallas import tpu_sc as plsc

N_FIELDS = 26
VOCAB = 100000
EMBED_DIM = 32
BATCH = 4096

_NC = 2
_NS = 16
_NW = _NC * _NS
_BPW = BATCH // _NW
_IPW = _BPW * N_FIELDS
_ICHUNK = 128
_NCHUNKS = _IPW // _ICHUNK


def _sc_body(x_hbm, tab_hbm, out_hbm, xv, idxv, rows, outv, sem):
    wid = lax.axis_index("s") * _NC + lax.axis_index("c")
    base = wid * _IPW

    pltpu.sync_copy(x_hbm.at[pl.ds(base, _IPW)], xv)

    lane = lax.broadcasted_iota(jnp.int32, (16,), 0)

    def mk_idx(j, _):
        pos = j * 16 + lane
        f = lax.rem(pos, N_FIELDS)
        idxv[pl.ds(j * 16, 16)] = xv[pl.ds(j * 16, 16)] + f * VOCAB
        return 0

    lax.fori_loop(0, _IPW // 16, mk_idx, 0, unroll=False)

    copies = []
    for c in range(_NCHUNKS):
        copies.append(
            pltpu.async_copy(
                tab_hbm.at[idxv.at[pl.ds(c * _ICHUNK, _ICHUNK)]],
                rows.at[pl.ds(c * _ICHUNK, _ICHUNK)],
                sem,
            )
        )
    for c in copies:
        c.wait()

    def acc_row(i, _):
        r = i * N_FIELDS
        lo = rows[r, 0:16]
        hi = rows[r, 16:32]
        for f in range(1, N_FIELDS):
            lo = lo + rows[r + f, 0:16]
            hi = hi + rows[r + f, 16:32]
        outv[pl.ds(i * EMBED_DIM, 16)] = lo
        outv[pl.ds(i * EMBED_DIM + 16, 16)] = hi
        return 0

    lax.fori_loop(0, _BPW, acc_row, 0, unroll=False)

    pltpu.sync_copy(outv, out_hbm.at[pl.ds(wid * _BPW * EMBED_DIM, _BPW * EMBED_DIM)])


@jax.jit
def _sc_embed_sum(x_flat, tab_flat):
    mesh = plsc.VectorSubcoreMesh(core_axis_name="c", subcore_axis_name="s")
    k = functools.partial(
        pl.kernel,
        mesh=mesh,
        out_type=jax.ShapeDtypeStruct((BATCH * EMBED_DIM,), jnp.float32),
        scratch_types=[
            pltpu.VMEM((_IPW,), jnp.int32),
            pltpu.VMEM((_IPW,), jnp.int32),
            pltpu.VMEM((_IPW, EMBED_DIM), jnp.float32),
            pltpu.VMEM((_BPW * EMBED_DIM,), jnp.float32),
            pltpu.SemaphoreType.DMA,
        ],
        compiler_params=pltpu.CompilerParams(use_tc_tiling_on_sc=False),
    )(_sc_body)
    return k(x_flat, tab_flat)


def kernel(x, tables):
    x_flat = x.astype(jnp.int32).reshape(BATCH * N_FIELDS)
    tab_flat = tables.reshape(N_FIELDS * VOCAB, EMBED_DIM)
    out = _sc_embed_sum(x_flat, tab_flat)
    return out.reshape(BATCH, EMBED_DIM)


# speedup vs baseline: 1.0005x; 1.0005x over previous
import functools

import jax
import jax.numpy as jnp
from jax import lax
from jax.experimental import pallas as pl
from jax.experimental.pallas import tpu as pltpu
from jax.experimental.pallas import tpu_sc as plsc

N_FIELDS = 26
VOCAB = 100000
EMBED_DIM = 32
BATCH = 4096

_NC = 2
_NS = 16
_NW = _NC * _NS
_BPW = BATCH // _NW
_IPW = _BPW * N_FIELDS
_ICHUNK = 128
_NCHUNKS = _IPW // _ICHUNK


def _sc_body(x_hbm, tab_hbm, out_hbm, xv, idxv, rows, outv, sem):
    wid = lax.axis_index("s") * _NC + lax.axis_index("c")
    base = wid * _IPW

    pltpu.sync_copy(x_hbm.at[pl.ds(base, _IPW)], xv)

    lane = lax.broadcasted_iota(jnp.int32, (16,), 0)

    def mk_idx(j, _):
        pos = j * 16 + lane
        f = lax.rem(pos, N_FIELDS)
        idxv[pl.ds(j * 16, 16)] = xv[pl.ds(j * 16, 16)] + f * VOCAB
        return 0

    lax.fori_loop(0, _IPW // 16, mk_idx, 0, unroll=False)

    pltpu.async_copy(tab_hbm.at[idxv], rows, sem).wait()

    def acc_row(i, _):
        r = i * N_FIELDS
        lo = rows[r, 0:16]
        hi = rows[r, 16:32]
        for f in range(1, N_FIELDS):
            lo = lo + rows[r + f, 0:16]
            hi = hi + rows[r + f, 16:32]
        outv[pl.ds(i * EMBED_DIM, 16)] = lo
        outv[pl.ds(i * EMBED_DIM + 16, 16)] = hi
        return 0

    lax.fori_loop(0, _BPW, acc_row, 0, unroll=False)

    pltpu.sync_copy(outv, out_hbm.at[pl.ds(wid * _BPW * EMBED_DIM, _BPW * EMBED_DIM)])


@jax.jit
def _sc_embed_sum(x_flat, tab_flat):
    mesh = plsc.VectorSubcoreMesh(core_axis_name="c", subcore_axis_name="s")
    k = functools.partial(
        pl.kernel,
        mesh=mesh,
        out_type=jax.ShapeDtypeStruct((BATCH * EMBED_DIM,), jnp.float32),
        scratch_types=[
            pltpu.VMEM((_IPW,), jnp.int32),
            pltpu.VMEM((_IPW,), jnp.int32),
            pltpu.VMEM((_IPW, EMBED_DIM), jnp.float32),
            pltpu.VMEM((_BPW * EMBED_DIM,), jnp.float32),
            pltpu.SemaphoreType.DMA,
        ],
        compiler_params=pltpu.CompilerParams(use_tc_tiling_on_sc=False),
    )(_sc_body)
    return k(x_flat, tab_flat)


def kernel(x, tables):
    x_flat = x.astype(jnp.int32).reshape(BATCH * N_FIELDS)
    tab_flat = tables.reshape(N_FIELDS * VOCAB, EMBED_DIM)
    out = _sc_embed_sum(x_flat, tab_flat)
    return out.reshape(BATCH, EMBED_DIM)
